# speedup vs baseline: 2.5824x; 2.5824x over previous
import functools

import jax
import jax.numpy as jnp
from jax import lax
from jax.experimental import pallas as pl
from jax.experimental.pallas import tpu as pltpu
from jax.experimental.pallas import tpu_sc as plsc

_B, _S = 4, 2048
_N = _B * _S
_HIDDEN = 768
_DDIM = 128
_NW = 32
_TPW = _N // _NW
_WCHUNK = 64
_EPS = 1e-12

_TOK_BLK = 256
_GRID = _N // _TOK_BLK
_S_BLKS = _S // _TOK_BLK


def _sc_gather(word_ids, domain_ids, word_emb, domain_emb):
    mesh = plsc.VectorSubcoreMesh(core_axis_name="c", subcore_axis_name="s")

    @functools.partial(
        pl.kernel,
        out_type=[
            jax.ShapeDtypeStruct((_N, _HIDDEN), jnp.float32),
            jax.ShapeDtypeStruct((_N, _DDIM), jnp.float32),
        ],
        mesh=mesh,
        scratch_types=[
            pltpu.VMEM((_TPW,), jnp.int32),
            pltpu.VMEM((_TPW,), jnp.int32),
            pltpu.VMEM((_WCHUNK, _HIDDEN), jnp.float32),
            pltpu.VMEM((_TPW, _DDIM), jnp.float32),
            pltpu.SemaphoreType.DMA,
        ],
    )
    def k(wids_hbm, dids_hbm, wtab_hbm, dtab_hbm, ow_hbm, od_hbm,
          widx_v, didx_v, wrows_v, drows_v, sem):
        wid = lax.axis_index("s") * 2 + lax.axis_index("c")
        base = wid * _TPW
        pltpu.sync_copy(wids_hbm.at[pl.ds(base, _TPW)], widx_v)
        pltpu.sync_copy(dids_hbm.at[pl.ds(base, _TPW)], didx_v)
        pltpu.async_copy(dtab_hbm.at[didx_v], drows_v, sem).wait()
        pltpu.sync_copy(drows_v, od_hbm.at[pl.ds(base, _TPW)])
        for c in range(_TPW // _WCHUNK):
            pltpu.async_copy(
                wtab_hbm.at[widx_v.at[pl.ds(c * _WCHUNK, _WCHUNK)]],
                wrows_v, sem).wait()
            pltpu.sync_copy(wrows_v, ow_hbm.at[pl.ds(base + c * _WCHUNK, _WCHUNK)])

    return k(word_ids, domain_ids, word_emb, domain_emb)


def _tc_body(gw_ref, gd_ref, pos_ref, tt_ref, w_ref, b_ref, g_ref, bb_ref,
             o_ref):
    proj = lax.dot_general(
        gd_ref[...], w_ref[...], (((1,), (0,)), ((), ())),
        precision=lax.Precision.HIGHEST, preferred_element_type=jnp.float32)
    e = gw_ref[...] + proj + pos_ref[...] + (tt_ref[0, :] + b_ref[0, :])[None, :]
    mean = jnp.mean(e, axis=1, keepdims=True)
    c = e - mean
    var = jnp.mean(c * c, axis=1, keepdims=True)
    o_ref[...] = (c * lax.rsqrt(var + _EPS)) * g_ref[0, :][None, :] \
        + bb_ref[0, :][None, :]


def _tc_fuse(gw, gd, pos_emb, type_row, W_proj, b_proj, ln_gamma, ln_beta):
    return pl.pallas_call(
        _tc_body,
        grid=(_GRID,),
        in_specs=[
            pl.BlockSpec((_TOK_BLK, _HIDDEN), lambda i: (i, 0)),
            pl.BlockSpec((_TOK_BLK, _DDIM), lambda i: (i, 0)),
            pl.BlockSpec((_TOK_BLK, _HIDDEN), lambda i: (i % _S_BLKS, 0)),
            pl.BlockSpec((1, _HIDDEN), lambda i: (0, 0)),
            pl.BlockSpec((_DDIM, _HIDDEN), lambda i: (0, 0)),
            pl.BlockSpec((1, _HIDDEN), lambda i: (0, 0)),
            pl.BlockSpec((1, _HIDDEN), lambda i: (0, 0)),
            pl.BlockSpec((1, _HIDDEN), lambda i: (0, 0)),
        ],
        out_specs=pl.BlockSpec((_TOK_BLK, _HIDDEN), lambda i: (i, 0)),
        out_shape=jax.ShapeDtypeStruct((_N, _HIDDEN), jnp.float32),
    )(gw, gd, pos_emb, type_row, W_proj, b_proj, ln_gamma, ln_beta)


def kernel(input_ids, domain_ids, word_emb, domain_emb, pos_emb, type_emb,
           W_proj, b_proj, ln_gamma, ln_beta):
    wids = input_ids.reshape(-1).astype(jnp.int32)
    dids = domain_ids.reshape(-1).astype(jnp.int32)
    gw, gd = _sc_gather(wids, dids, word_emb, domain_emb)
    out = _tc_fuse(gw, gd, pos_emb, type_emb[0:1], W_proj,
                   b_proj[None, :], ln_gamma[None, :], ln_beta[None, :])
    return out.reshape(_B, _S, _HIDDEN)


# speedup vs baseline: 2.9576x; 1.1453x over previous
import functools

import jax
import jax.numpy as jnp
from jax import lax
from jax.experimental import pallas as pl
from jax.experimental.pallas import tpu as pltpu
from jax.experimental.pallas import tpu_sc as plsc

_B, _S = 4, 2048
_N = _B * _S
_HIDDEN = 768
_DDIM = 128
_NW = 32
_TPW = _N // _NW
_WCHUNK = 64
_EPS = 1e-12

_TOK_BLK = 512
_S_BLKS = _S // _TOK_BLK


def _sc_gather(word_ids, domain_ids, word_emb, domain_emb):
    mesh = plsc.VectorSubcoreMesh(core_axis_name="c", subcore_axis_name="s")

    @functools.partial(
        pl.kernel,
        out_type=[
            jax.ShapeDtypeStruct((_N, _HIDDEN), jnp.float32),
            jax.ShapeDtypeStruct((_N, _DDIM), jnp.float32),
        ],
        mesh=mesh,
        scratch_types=[
            pltpu.VMEM((_TPW,), jnp.int32),
            pltpu.VMEM((_TPW,), jnp.int32),
            pltpu.VMEM((_WCHUNK, _HIDDEN), jnp.float32),
            pltpu.VMEM((_TPW, _DDIM), jnp.float32),
            pltpu.SemaphoreType.DMA,
        ],
    )
    def k(wids_hbm, dids_hbm, wtab_hbm, dtab_hbm, ow_hbm, od_hbm,
          widx_v, didx_v, wrows_v, drows_v, sem):
        wid = lax.axis_index("s") * 2 + lax.axis_index("c")
        base = wid * _TPW
        pltpu.sync_copy(wids_hbm.at[pl.ds(base, _TPW)], widx_v)
        pltpu.sync_copy(dids_hbm.at[pl.ds(base, _TPW)], didx_v)
        pltpu.async_copy(dtab_hbm.at[didx_v], drows_v, sem).wait()
        pltpu.sync_copy(drows_v, od_hbm.at[pl.ds(base, _TPW)])
        for c in range(_TPW // _WCHUNK):
            pltpu.async_copy(
                wtab_hbm.at[widx_v.at[pl.ds(c * _WCHUNK, _WCHUNK)]],
                wrows_v, sem).wait()
            pltpu.sync_copy(wrows_v, ow_hbm.at[pl.ds(base + c * _WCHUNK, _WCHUNK)])

    return k(word_ids, domain_ids, word_emb, domain_emb)


def _tc_body(gw_ref, gd_ref, pos_ref, cr_ref, w_ref, g_ref, bb_ref, o_ref):
    proj = lax.dot_general(
        gd_ref[...], w_ref[...], (((1,), (0,)), ((), ())),
        precision=lax.Precision.HIGHEST, preferred_element_type=jnp.float32)
    e = gw_ref[...] + proj + pos_ref[...] + cr_ref[0, :][None, :]
    mean = jnp.mean(e, axis=1, keepdims=True)
    c = e - mean
    var = jnp.mean(c * c, axis=1, keepdims=True)
    o_ref[...] = (c * lax.rsqrt(var + _EPS)) * g_ref[0, :][None, :] \
        + bb_ref[0, :][None, :]


def _tc_fuse(gw, gd, pos_emb, const_row, W_proj, ln_gamma, ln_beta):
    return pl.pallas_call(
        _tc_body,
        grid=(_S_BLKS, _B),
        in_specs=[
            pl.BlockSpec((_TOK_BLK, _HIDDEN), lambda i, j: (j * _S_BLKS + i, 0)),
            pl.BlockSpec((_TOK_BLK, _DDIM), lambda i, j: (j * _S_BLKS + i, 0)),
            pl.BlockSpec((_TOK_BLK, _HIDDEN), lambda i, j: (i, 0)),
            pl.BlockSpec((1, _HIDDEN), lambda i, j: (0, 0)),
            pl.BlockSpec((_DDIM, _HIDDEN), lambda i, j: (0, 0)),
            pl.BlockSpec((1, _HIDDEN), lambda i, j: (0, 0)),
            pl.BlockSpec((1, _HIDDEN), lambda i, j: (0, 0)),
        ],
        out_specs=pl.BlockSpec((_TOK_BLK, _HIDDEN),
                               lambda i, j: (j * _S_BLKS + i, 0)),
        out_shape=jax.ShapeDtypeStruct((_N, _HIDDEN), jnp.float32),
    )(gw, gd, pos_emb, const_row, W_proj, ln_gamma, ln_beta)


def kernel(input_ids, domain_ids, word_emb, domain_emb, pos_emb, type_emb,
           W_proj, b_proj, ln_gamma, ln_beta):
    wids = input_ids.reshape(-1).astype(jnp.int32)
    dids = domain_ids.reshape(-1).astype(jnp.int32)
    const_row = (type_emb[0] + b_proj)[None, :]
    gw, gd = _sc_gather(wids, dids, word_emb, domain_emb)
    out = _tc_fuse(gw, gd, pos_emb, const_row, W_proj,
                   ln_gamma[None, :], ln_beta[None, :])
    return out.reshape(_B, _S, _HIDDEN)


# speedup vs baseline: 3.4851x; 1.1784x over previous
import functools

import jax
import jax.numpy as jnp
from jax import lax
from jax.experimental import pallas as pl
from jax.experimental.pallas import tpu as pltpu
from jax.experimental.pallas import tpu_sc as plsc

_B, _S = 4, 2048
_N = _B * _S
_HIDDEN = 768
_DDIM = 128
_NW = 32
_TPW = _N // _NW
_WCHUNK = 64
_EPS = 1e-12

_TOK_BLK = 1024
_S_BLKS = _S // _TOK_BLK


def _sc_gather(word_ids, domain_ids, word_emb, domain_emb):
    mesh = plsc.VectorSubcoreMesh(core_axis_name="c", subcore_axis_name="s")

    @functools.partial(
        pl.kernel,
        out_type=[
            jax.ShapeDtypeStruct((_N, _HIDDEN), jnp.float32),
            jax.ShapeDtypeStruct((_N, _DDIM), jnp.float32),
        ],
        mesh=mesh,
        scratch_types=[
            pltpu.VMEM((_TPW,), jnp.int32),
            pltpu.VMEM((_TPW,), jnp.int32),
            pltpu.VMEM((_WCHUNK, _HIDDEN), jnp.float32),
            pltpu.VMEM((_TPW, _DDIM), jnp.float32),
            pltpu.SemaphoreType.DMA,
        ],
    )
    def k(wids_hbm, dids_hbm, wtab_hbm, dtab_hbm, ow_hbm, od_hbm,
          widx_v, didx_v, wrows_v, drows_v, sem):
        wid = lax.axis_index("s") * 2 + lax.axis_index("c")
        base = wid * _TPW
        pltpu.sync_copy(wids_hbm.at[pl.ds(base, _TPW)], widx_v)
        pltpu.sync_copy(dids_hbm.at[pl.ds(base, _TPW)], didx_v)
        pltpu.async_copy(dtab_hbm.at[didx_v], drows_v, sem).wait()
        pltpu.sync_copy(drows_v, od_hbm.at[pl.ds(base, _TPW)])
        for c in range(_TPW // _WCHUNK):
            pltpu.async_copy(
                wtab_hbm.at[widx_v.at[pl.ds(c * _WCHUNK, _WCHUNK)]],
                wrows_v, sem).wait()
            pltpu.sync_copy(wrows_v, ow_hbm.at[pl.ds(base + c * _WCHUNK, _WCHUNK)])

    return k(word_ids, domain_ids, word_emb, domain_emb)


def _tc_body(gw_ref, gd_ref, pos_ref, cr_ref, w_ref, g_ref, bb_ref, o_ref):
    proj = lax.dot_general(
        gd_ref[...], w_ref[...], (((1,), (0,)), ((), ())),
        precision=lax.Precision.DEFAULT, preferred_element_type=jnp.float32)
    e = gw_ref[...] + proj + pos_ref[...] + cr_ref[0, :][None, :]
    mean = jnp.mean(e, axis=1, keepdims=True)
    c = e - mean
    var = jnp.mean(c * c, axis=1, keepdims=True)
    o_ref[...] = (c * lax.rsqrt(var + _EPS)) * g_ref[0, :][None, :] \
        + bb_ref[0, :][None, :]


def _tc_fuse(gw, gd, pos_emb, const_row, W_proj, ln_gamma, ln_beta):
    return pl.pallas_call(
        _tc_body,
        grid=(_S_BLKS, _B),
        in_specs=[
            pl.BlockSpec((_TOK_BLK, _HIDDEN), lambda i, j: (j * _S_BLKS + i, 0)),
            pl.BlockSpec((_TOK_BLK, _DDIM), lambda i, j: (j * _S_BLKS + i, 0)),
            pl.BlockSpec((_TOK_BLK, _HIDDEN), lambda i, j: (i, 0)),
            pl.BlockSpec((1, _HIDDEN), lambda i, j: (0, 0)),
            pl.BlockSpec((_DDIM, _HIDDEN), lambda i, j: (0, 0)),
            pl.BlockSpec((1, _HIDDEN), lambda i, j: (0, 0)),
            pl.BlockSpec((1, _HIDDEN), lambda i, j: (0, 0)),
        ],
        out_specs=pl.BlockSpec((_TOK_BLK, _HIDDEN),
                               lambda i, j: (j * _S_BLKS + i, 0)),
        out_shape=jax.ShapeDtypeStruct((_N, _HIDDEN), jnp.float32),
    )(gw, gd, pos_emb, const_row, W_proj, ln_gamma, ln_beta)


def kernel(input_ids, domain_ids, word_emb, domain_emb, pos_emb, type_emb,
           W_proj, b_proj, ln_gamma, ln_beta):
    wids = input_ids.reshape(-1).astype(jnp.int32)
    dids = domain_ids.reshape(-1).astype(jnp.int32)
    const_row = (type_emb[0] + b_proj)[None, :]
    gw, gd = _sc_gather(wids, dids, word_emb, domain_emb)
    out = _tc_fuse(gw, gd, pos_emb, const_row, W_proj,
                   ln_gamma[None, :], ln_beta[None, :])
    return out.reshape(_B, _S, _HIDDEN)


# speedup vs baseline: 3.6141x; 1.0370x over previous
import functools

import jax
import jax.numpy as jnp
from jax import lax
from jax.experimental import pallas as pl
from jax.experimental.pallas import tpu as pltpu
from jax.experimental.pallas import tpu_sc as plsc

_B, _S = 4, 2048
_N = _B * _S
_HIDDEN = 768
_DDIM = 128
_NW = 32
_TPW = _N // _NW
_WCHUNK = 64
_EPS = 1e-12

_TOK_BLK = 1024
_S_BLKS = _S // _TOK_BLK


def _sc_gather(word_ids, domain_ids, word_emb, domain_emb):
    mesh = plsc.VectorSubcoreMesh(core_axis_name="c", subcore_axis_name="s")

    dchunk = _TPW // 2

    @functools.partial(
        pl.kernel,
        out_type=[
            jax.ShapeDtypeStruct((_N, _HIDDEN), jnp.float32),
            jax.ShapeDtypeStruct((_N, _DDIM), jnp.float32),
        ],
        mesh=mesh,
        scratch_types=[
            pltpu.VMEM((_TPW,), jnp.int32),
            pltpu.VMEM((_TPW,), jnp.int32),
            pltpu.VMEM((_WCHUNK, _HIDDEN), jnp.float32),
            pltpu.VMEM((_WCHUNK, _HIDDEN), jnp.float32),
            pltpu.VMEM((dchunk, _DDIM), jnp.float32),
            pltpu.SemaphoreType.DMA,
            pltpu.SemaphoreType.DMA,
            pltpu.SemaphoreType.DMA,
            pltpu.SemaphoreType.DMA,
            pltpu.SemaphoreType.DMA,
            pltpu.SemaphoreType.DMA,
            pltpu.SemaphoreType.DMA,
        ],
    )
    def k(wids_hbm, dids_hbm, wtab_hbm, dtab_hbm, ow_hbm, od_hbm,
          widx_v, didx_v, wb0, wb1, db, sI, sW0, sW1, sC0, sC1, sD, sCD):
        wid = lax.axis_index("s") * 2 + lax.axis_index("c")
        base = wid * _TPW
        wb = (wb0, wb1)
        sW = (sW0, sW1)
        sC = (sC0, sC1)
        ci0 = pltpu.async_copy(wids_hbm.at[pl.ds(base, _TPW)], widx_v, sI)
        ci1 = pltpu.async_copy(dids_hbm.at[pl.ds(base, _TPW)], didx_v, sI)
        ci0.wait()
        ci1.wait()
        nw = _TPW // _WCHUNK
        g = [None] * nw
        co = [None] * nw
        g[0] = pltpu.async_copy(
            wtab_hbm.at[widx_v.at[pl.ds(0, _WCHUNK)]], wb[0], sW[0])
        g[1] = pltpu.async_copy(
            wtab_hbm.at[widx_v.at[pl.ds(_WCHUNK, _WCHUNK)]], wb[1], sW[1])
        gd = pltpu.async_copy(dtab_hbm.at[didx_v.at[pl.ds(0, dchunk)]], db, sD)
        g[0].wait()
        co[0] = pltpu.async_copy(wb[0], ow_hbm.at[pl.ds(base, _WCHUNK)], sC[0])
        g[1].wait()
        co[1] = pltpu.async_copy(
            wb[1], ow_hbm.at[pl.ds(base + _WCHUNK, _WCHUNK)], sC[1])
        for c in range(2, nw):
            co[c - 2].wait()
            g[c] = pltpu.async_copy(
                wtab_hbm.at[widx_v.at[pl.ds(c * _WCHUNK, _WCHUNK)]],
                wb[c % 2], sW[c % 2])
        gd.wait()
        cd0 = pltpu.async_copy(db, od_hbm.at[pl.ds(base, dchunk)], sCD)
        for c in range(2, nw):
            g[c].wait()
            co[c] = pltpu.async_copy(
                wb[c % 2], ow_hbm.at[pl.ds(base + c * _WCHUNK, _WCHUNK)],
                sC[c % 2])
        cd0.wait()
        gd1 = pltpu.async_copy(
            dtab_hbm.at[didx_v.at[pl.ds(dchunk, dchunk)]], db, sD)
        gd1.wait()
        cd1 = pltpu.async_copy(db, od_hbm.at[pl.ds(base + dchunk, dchunk)], sCD)
        for c in range(nw - 2, nw):
            co[c].wait()
        cd1.wait()

    return k(word_ids, domain_ids, word_emb, domain_emb)


def _tc_body(gw_ref, gd_ref, pos_ref, cr_ref, w_ref, g_ref, bb_ref, o_ref):
    proj = lax.dot_general(
        gd_ref[...], w_ref[...], (((1,), (0,)), ((), ())),
        precision=lax.Precision.DEFAULT, preferred_element_type=jnp.float32)
    e = gw_ref[...] + proj + pos_ref[...] + cr_ref[0, :][None, :]
    mean = jnp.mean(e, axis=1, keepdims=True)
    c = e - mean
    var = jnp.mean(c * c, axis=1, keepdims=True)
    o_ref[...] = (c * lax.rsqrt(var + _EPS)) * g_ref[0, :][None, :] \
        + bb_ref[0, :][None, :]


def _tc_fuse(gw, gd, pos_emb, const_row, W_proj, ln_gamma, ln_beta):
    return pl.pallas_call(
        _tc_body,
        grid=(_S_BLKS, _B),
        in_specs=[
            pl.BlockSpec((_TOK_BLK, _HIDDEN), lambda i, j: (j * _S_BLKS + i, 0)),
            pl.BlockSpec((_TOK_BLK, _DDIM), lambda i, j: (j * _S_BLKS + i, 0)),
            pl.BlockSpec((_TOK_BLK, _HIDDEN), lambda i, j: (i, 0)),
            pl.BlockSpec((1, _HIDDEN), lambda i, j: (0, 0)),
            pl.BlockSpec((_DDIM, _HIDDEN), lambda i, j: (0, 0)),
            pl.BlockSpec((1, _HIDDEN), lambda i, j: (0, 0)),
            pl.BlockSpec((1, _HIDDEN), lambda i, j: (0, 0)),
        ],
        out_specs=pl.BlockSpec((_TOK_BLK, _HIDDEN),
                               lambda i, j: (j * _S_BLKS + i, 0)),
        out_shape=jax.ShapeDtypeStruct((_N, _HIDDEN), jnp.float32),
    )(gw, gd, pos_emb, const_row, W_proj, ln_gamma, ln_beta)


def kernel(input_ids, domain_ids, word_emb, domain_emb, pos_emb, type_emb,
           W_proj, b_proj, ln_gamma, ln_beta):
    wids = input_ids.reshape(-1).astype(jnp.int32)
    dids = domain_ids.reshape(-1).astype(jnp.int32)
    const_row = (type_emb[0] + b_proj)[None, :]
    gw, gd = _sc_gather(wids, dids, word_emb, domain_emb)
    out = _tc_fuse(gw, gd, pos_emb, const_row, W_proj,
                   ln_gamma[None, :], ln_beta[None, :])
    return out.reshape(_B, _S, _HIDDEN)
